# baseline (speedup 1.0000x reference)
import functools

import jax
import jax.numpy as jnp
from jax import lax
from jax.experimental import pallas as pl
from jax.experimental.pallas import tpu as pltpu
from jax.experimental.pallas import tpu_sc as plsc

NC = 2
NS = 16
NW = NC * NS
LANES = 16
B = 128


def _sc_mesh():
    return plsc.VectorSubcoreMesh(
        core_axis_name="c", subcore_axis_name="s",
        num_cores=NC, num_subcores=NS)


def _make_degree_kernel(n_acc2, rows_per_tile):
    per_tile = n_acc2 // NS

    @functools.partial(
        pl.kernel,
        out_type=jax.ShapeDtypeStruct((NC, n_acc2), jnp.float32),
        mesh=_sc_mesh(),
        scratch_types=[
            pltpu.VMEM((rows_per_tile, B), jnp.int32),
            pltpu.VMEM((B,), jnp.float32),
            pltpu.VMEM((B,), jnp.float32),
            pltpu.VMEM_SHARED((n_acc2,), jnp.float32),
            pltpu.SemaphoreType.DMA,
        ],
    )
    def deg_kernel(ddst_hbm, ones_hbm, out_hbm, didx_v, ones_v, z_v,
                   acc_sh, sem):
        cid = lax.axis_index("c")
        sid = lax.axis_index("s")
        wid = cid * NS + sid
        zf = jnp.zeros((LANES,), jnp.float32)
        for i in range(B // LANES):
            z_v[pl.ds(i * LANES, LANES)] = zf

        @pl.loop(0, per_tile // B)
        def _zero(t):
            pltpu.sync_copy(z_v, acc_sh.at[pl.ds(sid * per_tile + t * B, B)])

        pltpu.async_copy(
            ddst_hbm.at[pl.ds(wid * rows_per_tile, rows_per_tile)],
            didx_v, sem).wait()
        pltpu.sync_copy(ones_hbm, ones_v)
        plsc.subcore_barrier()

        @pl.loop(0, rows_per_tile)
        def _scatter(j):
            pltpu.async_copy(ones_v, acc_sh.at[didx_v.at[j]], sem,
                             add=True)

            @pl.when(j >= 16)
            def _():
                pltpu.make_async_copy(ones_v, acc_sh.at[didx_v.at[0]],
                                      sem).wait()

        @pl.loop(0, 16)
        def _drain(j):
            pltpu.make_async_copy(ones_v, acc_sh.at[didx_v.at[0]],
                                  sem).wait()

        plsc.subcore_barrier()
        pltpu.sync_copy(acc_sh.at[pl.ds(sid * per_tile, per_tile)],
                        out_hbm.at[cid, pl.ds(sid * per_tile, per_tile)])

    return deg_kernel


def _make_edge_kernel(n, d, n_acc, rows_per_tile):
    acc_rows = n_acc // NS
    assert rows_per_tile % 2 == 0
    win = rows_per_tile // 2

    @functools.partial(
        pl.kernel,
        out_type=jax.ShapeDtypeStruct((NC, n_acc, d), jnp.float32),
        mesh=_sc_mesh(),
        scratch_types=[
            pltpu.VMEM((win, B), jnp.int32),
            pltpu.VMEM((win, B), jnp.int32),
            pltpu.VMEM((2, B, d), jnp.float32),
            pltpu.VMEM_SHARED((n_acc, d), jnp.float32),
            pltpu.SemaphoreType.DMA,
            pltpu.SemaphoreType.DMA,
            pltpu.SemaphoreType.DMA,
            pltpu.SemaphoreType.DMA,
        ],
    )
    def edge_kernel(src_hbm, dst_hbm, y_hbm, out_hbm, sidx_v, didx_v,
                    rows_v, acc_sh, gsem0, gsem1, ssem0, ssem1):
        cid = lax.axis_index("c")
        sid = lax.axis_index("s")
        wid = cid * NS + sid
        zf = jnp.zeros((LANES,), jnp.float32)
        for r in range(LANES):
            for i in range(d // LANES):
                rows_v[0, r, pl.ds(i * LANES, LANES)] = zf

        @pl.loop(0, acc_rows // LANES)
        def _zero(t):
            pltpu.sync_copy(
                rows_v.at[0, pl.ds(0, LANES)],
                acc_sh.at[pl.ds(sid * acc_rows + t * LANES, LANES)])

        plsc.subcore_barrier()

        gsems = (gsem0, gsem1)
        ssems = (ssem0, ssem1)
        for half in range(2):
            base = wid * rows_per_tile + half * win
            pltpu.async_copy(src_hbm.at[pl.ds(base, win)], sidx_v,
                             gsem0).wait()
            pltpu.async_copy(dst_hbm.at[pl.ds(base, win)], didx_v,
                             gsem0).wait()
            pltpu.async_copy(y_hbm.at[sidx_v.at[0]], rows_v.at[0],
                             gsem0)

            @pl.loop(0, win)
            def _edges(j):
                b = lax.rem(j, 2)
                for k in range(2):
                    @pl.when(b == k)
                    def _(k=k):
                        ko = 1 - k

                        @pl.when(j >= 1)
                        def _():
                            pltpu.make_async_copy(
                                rows_v.at[ko],
                                acc_sh.at[didx_v.at[j - 1]],
                                ssems[ko]).wait()

                        @pl.when(j + 1 < win)
                        def _():
                            pltpu.async_copy(y_hbm.at[sidx_v.at[j + 1]],
                                             rows_v.at[ko], gsems[ko])
                        pltpu.make_async_copy(
                            y_hbm.at[sidx_v.at[j]], rows_v.at[k],
                            gsems[k]).wait()
                        pltpu.async_copy(rows_v.at[k],
                                         acc_sh.at[didx_v.at[j]],
                                         ssems[k], add=True)

            lk = (win - 1) % 2
            pltpu.make_async_copy(rows_v.at[lk],
                                  acc_sh.at[didx_v.at[win - 1]],
                                  ssems[lk]).wait()
        plsc.subcore_barrier()
        pltpu.sync_copy(acc_sh.at[pl.ds(sid * acc_rows, acc_rows)],
                        out_hbm.at[cid, pl.ds(sid * acc_rows, acc_rows)])

    return edge_kernel


def _tc_matmul_scale(h, w, dp0, dp1, block_rows):
    n, d = h.shape

    def body(h_ref, w_ref, dp0_ref, dp1_ref, y_ref):
        dis = lax.rsqrt(1.0 + dp0_ref[...] + dp1_ref[...])
        xw = jnp.dot(h_ref[...], w_ref[...],
                     preferred_element_type=jnp.float32)
        y_ref[...] = xw * dis

    return pl.pallas_call(
        body,
        grid=(n // block_rows,),
        in_specs=[
            pl.BlockSpec((block_rows, d), lambda i: (i, 0)),
            pl.BlockSpec((d, d), lambda i: (0, 0)),
            pl.BlockSpec((block_rows, 1), lambda i: (i, 0)),
            pl.BlockSpec((block_rows, 1), lambda i: (i, 0)),
        ],
        out_specs=pl.BlockSpec((block_rows, d), lambda i: (i, 0)),
        out_shape=jax.ShapeDtypeStruct((n, d), jnp.float32),
    )(h, w, dp0, dp1)


def _tc_finalize_matmul(parts, y, dp0, dp1, b2d, w_next, dpn0, dpn1,
                        block_rows):
    n, d = y.shape

    def body(p0_ref, p1_ref, y_ref, dp0_ref, dp1_ref, b_ref, w_ref,
             dpn0_ref, dpn1_ref, o_ref):
        dis = lax.rsqrt(1.0 + dp0_ref[...] + dp1_ref[...])
        t = (p0_ref[0] + p1_ref[0] + y_ref[...]) * dis + b_ref[...]
        t = jnp.maximum(t, 0.0)
        nrm = jnp.sqrt(jnp.sum(t * t, axis=1, keepdims=True))
        h = t / jnp.maximum(nrm, 1e-12)
        disn = lax.rsqrt(1.0 + dpn0_ref[...] + dpn1_ref[...])
        o_ref[...] = jnp.dot(h, w_ref[...],
                             preferred_element_type=jnp.float32) * disn

    row_spec = pl.BlockSpec((block_rows, d), lambda i: (i, 0))
    col_spec = pl.BlockSpec((block_rows, 1), lambda i: (i, 0))
    part0_spec = pl.BlockSpec((1, block_rows, d), lambda i: (0, i, 0))
    part1_spec = pl.BlockSpec((1, block_rows, d), lambda i: (1, i, 0))
    return pl.pallas_call(
        body,
        grid=(n // block_rows,),
        in_specs=[
            part0_spec, part1_spec, row_spec, col_spec, col_spec,
            pl.BlockSpec((1, d), lambda i: (0, 0)),
            pl.BlockSpec((d, d), lambda i: (0, 0)),
            col_spec, col_spec,
        ],
        out_specs=row_spec,
        out_shape=jax.ShapeDtypeStruct((n, d), jnp.float32),
    )(parts, parts, y, dp0, dp1, b2d, w_next, dpn0, dpn1)


def _tc_finalize(parts, y, dp0, dp1, b2d, relu, block_rows):
    n, d = y.shape

    def body(p0_ref, p1_ref, y_ref, dp0_ref, dp1_ref, b_ref, o_ref):
        dis = lax.rsqrt(1.0 + dp0_ref[...] + dp1_ref[...])
        t = (p0_ref[0] + p1_ref[0] + y_ref[...]) * dis + b_ref[...]
        if relu:
            t = jnp.maximum(t, 0.0)
        nrm = jnp.sqrt(jnp.sum(t * t, axis=1, keepdims=True))
        o_ref[...] = t / jnp.maximum(nrm, 1e-12)

    return pl.pallas_call(
        body,
        grid=(n // block_rows,),
        in_specs=[
            pl.BlockSpec((1, block_rows, d), lambda i: (0, i, 0)),
            pl.BlockSpec((1, block_rows, d), lambda i: (1, i, 0)),
            pl.BlockSpec((block_rows, d), lambda i: (i, 0)),
            pl.BlockSpec((block_rows, 1), lambda i: (i, 0)),
            pl.BlockSpec((block_rows, 1), lambda i: (i, 0)),
            pl.BlockSpec((1, d), lambda i: (0, 0)),
        ],
        out_specs=pl.BlockSpec((block_rows, d), lambda i: (i, 0)),
        out_shape=jax.ShapeDtypeStruct((n, d), jnp.float32),
    )(parts, parts, y, dp0, dp1, b2d)


def kernel(x, edge_index_list, W0, b0, W1, b1):
    n, d = x.shape
    num_steps = edge_index_list.shape[0]
    e = edge_index_list.shape[2]

    n_acc = ((n + B) + 1023) // 1024 * 1024
    ept = -(-e // (NW * 1024)) * 1024
    e_pad = NW * ept
    pad_len = e_pad - e

    idx_dtype = edge_index_list.dtype
    pad_cycle = jnp.arange(pad_len, dtype=idx_dtype) % B
    pad_src = pad_cycle
    pad_dst = n + pad_cycle

    src2d = []
    dst2d = []
    deg_dst = []
    for s in range(num_steps):
        src_s = jnp.concatenate([edge_index_list[s, 0], pad_src])
        dst_s = jnp.concatenate([edge_index_list[s, 1], pad_dst])
        src2d.append(src_s.reshape(e_pad // B, B))
        dst2d.append(dst_s.reshape(e_pad // B, B))
        deg_dst.append(dst_s + s * n_acc)

    n_acc2 = num_steps * n_acc
    deg_rows_per_tile = num_steps * ept // B
    ddst2d = jnp.concatenate(deg_dst).reshape(num_steps * e_pad // B, B)
    ones_row = jnp.ones((B,), jnp.float32)

    deg_kernel = _make_degree_kernel(n_acc2, deg_rows_per_tile)
    degp = deg_kernel(ddst2d, ones_row)
    degp = degp.reshape(NC, num_steps, n_acc)

    edge_kernel = _make_edge_kernel(n, d, n_acc, ept // B)
    block_rows = 400

    dps = [(degp[0, s, :n].reshape(n, 1), degp[1, s, :n].reshape(n, 1))
           for s in range(num_steps)]

    y = _tc_matmul_scale(x, W0, dps[0][0], dps[0][1], block_rows)
    parts = edge_kernel(src2d[0], dst2d[0], y)
    y = _tc_finalize_matmul(parts, y, dps[0][0], dps[0][1],
                            b0.reshape(1, d), W1, dps[1][0], dps[1][1],
                            block_rows)
    parts = edge_kernel(src2d[1], dst2d[1], y)
    return _tc_finalize(parts, y, dps[1][0], dps[1][1],
                        b1.reshape(1, d), False, block_rows)


# speedup vs baseline: 1.0934x; 1.0934x over previous
import functools

import jax
import jax.numpy as jnp
from jax import lax
from jax.experimental import pallas as pl
from jax.experimental.pallas import tpu as pltpu
from jax.experimental.pallas import tpu_sc as plsc

NC = 2
NS = 16
NW = NC * NS
LANES = 16
B = 128


def _sc_mesh():
    return plsc.VectorSubcoreMesh(
        core_axis_name="c", subcore_axis_name="s",
        num_cores=NC, num_subcores=NS)


def _make_degree_kernel(n_acc2, rows_per_tile):
    per_tile = n_acc2 // NS

    @functools.partial(
        pl.kernel,
        out_type=jax.ShapeDtypeStruct((NC, n_acc2), jnp.float32),
        mesh=_sc_mesh(),
        scratch_types=[
            pltpu.VMEM((rows_per_tile, B), jnp.int32),
            pltpu.VMEM((B,), jnp.float32),
            pltpu.VMEM((B,), jnp.float32),
            pltpu.VMEM_SHARED((n_acc2,), jnp.float32),
            pltpu.SemaphoreType.DMA,
        ],
    )
    def deg_kernel(ddst_hbm, ones_hbm, out_hbm, didx_v, ones_v, z_v,
                   acc_sh, sem):
        cid = lax.axis_index("c")
        sid = lax.axis_index("s")
        wid = cid * NS + sid
        zf = jnp.zeros((LANES,), jnp.float32)
        for i in range(B // LANES):
            z_v[pl.ds(i * LANES, LANES)] = zf

        @pl.loop(0, per_tile // B)
        def _zero(t):
            pltpu.sync_copy(z_v, acc_sh.at[pl.ds(sid * per_tile + t * B, B)])

        pltpu.async_copy(
            ddst_hbm.at[pl.ds(wid * rows_per_tile, rows_per_tile)],
            didx_v, sem).wait()
        pltpu.sync_copy(ones_hbm, ones_v)
        plsc.subcore_barrier()

        @pl.loop(0, rows_per_tile)
        def _scatter(j):
            pltpu.async_copy(ones_v, acc_sh.at[didx_v.at[j]], sem,
                             add=True)

            @pl.when(j >= 16)
            def _():
                pltpu.make_async_copy(ones_v, acc_sh.at[didx_v.at[0]],
                                      sem).wait()

        @pl.loop(0, 16)
        def _drain(j):
            pltpu.make_async_copy(ones_v, acc_sh.at[didx_v.at[0]],
                                  sem).wait()

        plsc.subcore_barrier()
        pltpu.sync_copy(acc_sh.at[pl.ds(sid * per_tile, per_tile)],
                        out_hbm.at[cid, pl.ds(sid * per_tile, per_tile)])

    return deg_kernel


def _make_edge_kernel(n, d, n_acc, rows_per_tile):
    acc_rows = n_acc // NS
    assert rows_per_tile % 2 == 0
    win = rows_per_tile // 2

    @functools.partial(
        pl.kernel,
        out_type=jax.ShapeDtypeStruct((NC, n_acc, d), jnp.float32),
        mesh=_sc_mesh(),
        scratch_types=[
            pltpu.VMEM((win, B), jnp.int32),
            pltpu.VMEM((win, B), jnp.int32),
            pltpu.VMEM((2, B, d), jnp.float32),
            pltpu.VMEM_SHARED((n_acc, d), jnp.float32),
            pltpu.SemaphoreType.DMA,
            pltpu.SemaphoreType.DMA,
            pltpu.SemaphoreType.DMA,
            pltpu.SemaphoreType.DMA,
        ],
    )
    def edge_kernel(src_hbm, dst_hbm, y_hbm, out_hbm, sidx_v, didx_v,
                    rows_v, acc_sh, gsem0, gsem1, ssem0, ssem1):
        cid = lax.axis_index("c")
        sid = lax.axis_index("s")
        wid = cid * NS + sid
        zf = jnp.zeros((LANES,), jnp.float32)
        for r in range(LANES):
            for i in range(d // LANES):
                rows_v[0, r, pl.ds(i * LANES, LANES)] = zf

        @pl.loop(0, acc_rows // LANES)
        def _zero(t):
            pltpu.sync_copy(
                rows_v.at[0, pl.ds(0, LANES)],
                acc_sh.at[pl.ds(sid * acc_rows + t * LANES, LANES)])

        plsc.subcore_barrier()

        gsems = (gsem0, gsem1)
        ssems = (ssem0, ssem1)
        for half in range(2):
            base = wid * rows_per_tile + half * win
            pltpu.async_copy(src_hbm.at[pl.ds(base, win)], sidx_v,
                             gsem0).wait()
            pltpu.async_copy(dst_hbm.at[pl.ds(base, win)], didx_v,
                             gsem0).wait()
            pltpu.async_copy(y_hbm.at[sidx_v.at[0]], rows_v.at[0],
                             gsem0)

            @pl.loop(0, win)
            def _edges(j):
                b = lax.rem(j, 2)
                for k in range(2):
                    @pl.when(b == k)
                    def _(k=k):
                        ko = 1 - k

                        @pl.when(j >= 1)
                        def _():
                            pltpu.make_async_copy(
                                rows_v.at[ko],
                                acc_sh.at[didx_v.at[j - 1]],
                                ssems[ko]).wait()

                        @pl.when(j + 1 < win)
                        def _():
                            pltpu.async_copy(y_hbm.at[sidx_v.at[j + 1]],
                                             rows_v.at[ko], gsems[ko])
                        pltpu.make_async_copy(
                            y_hbm.at[sidx_v.at[j]], rows_v.at[k],
                            gsems[k]).wait()
                        pltpu.async_copy(rows_v.at[k],
                                         acc_sh.at[didx_v.at[j]],
                                         ssems[k], add=True)

            lk = (win - 1) % 2
            pltpu.make_async_copy(rows_v.at[lk],
                                  acc_sh.at[didx_v.at[win - 1]],
                                  ssems[lk]).wait()
        plsc.subcore_barrier()
        pltpu.sync_copy(acc_sh.at[pl.ds(sid * acc_rows, acc_rows)],
                        out_hbm.at[cid, pl.ds(sid * acc_rows, acc_rows)])

    return edge_kernel


def _tc_matmul_scale(h, w, dp0, dp1, block_rows):
    n, d = h.shape

    def body(h_ref, w_ref, dp0_ref, dp1_ref, y_ref):
        dis = lax.rsqrt(1.0 + dp0_ref[...] + dp1_ref[...])
        xw = jnp.dot(h_ref[...], w_ref[...],
                     preferred_element_type=jnp.float32)
        y_ref[...] = xw * dis

    return pl.pallas_call(
        body,
        grid=(n // block_rows,),
        in_specs=[
            pl.BlockSpec((block_rows, d), lambda i: (i, 0)),
            pl.BlockSpec((d, d), lambda i: (0, 0)),
            pl.BlockSpec((block_rows, 1), lambda i: (i, 0)),
            pl.BlockSpec((block_rows, 1), lambda i: (i, 0)),
        ],
        out_specs=pl.BlockSpec((block_rows, d), lambda i: (i, 0)),
        out_shape=jax.ShapeDtypeStruct((n, d), jnp.float32),
    )(h, w, dp0, dp1)


def _tc_finalize_matmul(parts, y, dp0, dp1, b2d, w_next, dpn0, dpn1,
                        block_rows):
    n, d = y.shape

    def body(p0_ref, p1_ref, y_ref, dp0_ref, dp1_ref, b_ref, w_ref,
             dpn0_ref, dpn1_ref, o_ref):
        dis = lax.rsqrt(1.0 + dp0_ref[...] + dp1_ref[...])
        t = (p0_ref[0] + p1_ref[0] + y_ref[...]) * dis + b_ref[...]
        t = jnp.maximum(t, 0.0)
        nrm = jnp.sqrt(jnp.sum(t * t, axis=1, keepdims=True))
        h = t / jnp.maximum(nrm, 1e-12)
        disn = lax.rsqrt(1.0 + dpn0_ref[...] + dpn1_ref[...])
        o_ref[...] = jnp.dot(h, w_ref[...],
                             preferred_element_type=jnp.float32) * disn

    row_spec = pl.BlockSpec((block_rows, d), lambda i: (i, 0))
    col_spec = pl.BlockSpec((block_rows, 1), lambda i: (i, 0))
    part0_spec = pl.BlockSpec((1, block_rows, d), lambda i: (0, i, 0))
    part1_spec = pl.BlockSpec((1, block_rows, d), lambda i: (1, i, 0))
    return pl.pallas_call(
        body,
        grid=(n // block_rows,),
        in_specs=[
            part0_spec, part1_spec, row_spec, col_spec, col_spec,
            pl.BlockSpec((1, d), lambda i: (0, 0)),
            pl.BlockSpec((d, d), lambda i: (0, 0)),
            col_spec, col_spec,
        ],
        out_specs=row_spec,
        out_shape=jax.ShapeDtypeStruct((n, d), jnp.float32),
    )(parts, parts, y, dp0, dp1, b2d, w_next, dpn0, dpn1)


def _tc_finalize(parts, y, dp0, dp1, b2d, relu, block_rows):
    n, d = y.shape

    def body(p0_ref, p1_ref, y_ref, dp0_ref, dp1_ref, b_ref, o_ref):
        dis = lax.rsqrt(1.0 + dp0_ref[...] + dp1_ref[...])
        t = (p0_ref[0] + p1_ref[0] + y_ref[...]) * dis + b_ref[...]
        if relu:
            t = jnp.maximum(t, 0.0)
        nrm = jnp.sqrt(jnp.sum(t * t, axis=1, keepdims=True))
        o_ref[...] = t / jnp.maximum(nrm, 1e-12)

    return pl.pallas_call(
        body,
        grid=(n // block_rows,),
        in_specs=[
            pl.BlockSpec((1, block_rows, d), lambda i: (0, i, 0)),
            pl.BlockSpec((1, block_rows, d), lambda i: (1, i, 0)),
            pl.BlockSpec((block_rows, d), lambda i: (i, 0)),
            pl.BlockSpec((block_rows, 1), lambda i: (i, 0)),
            pl.BlockSpec((block_rows, 1), lambda i: (i, 0)),
            pl.BlockSpec((1, d), lambda i: (0, 0)),
        ],
        out_specs=pl.BlockSpec((block_rows, d), lambda i: (i, 0)),
        out_shape=jax.ShapeDtypeStruct((n, d), jnp.float32),
    )(parts, parts, y, dp0, dp1, b2d)


def kernel(x, edge_index_list, W0, b0, W1, b1):
    n, d = x.shape
    num_steps = edge_index_list.shape[0]
    e = edge_index_list.shape[2]

    n_acc = ((n + B) + 1023) // 1024 * 1024
    ept = -(-e // (NW * 1024)) * 1024
    e_pad = NW * ept
    pad_len = e_pad - e

    idx_dtype = edge_index_list.dtype
    pad_cycle = jnp.arange(pad_len, dtype=idx_dtype) % B
    pad_src = pad_cycle
    pad_dst = n + pad_cycle

    src2d = []
    dst2d = []
    deg_dst = []
    for s in range(num_steps):
        src_s = jnp.concatenate([edge_index_list[s, 0], pad_src])
        dst_s = jnp.concatenate([edge_index_list[s, 1], pad_dst])
        src2d.append(src_s.reshape(e_pad // B, B))
        dst2d.append(dst_s.reshape(e_pad // B, B))
        deg_dst.append(dst_s + s * n_acc)

    n_acc2 = num_steps * n_acc
    deg_rows_per_tile = num_steps * ept // B
    ddst2d = jnp.concatenate(deg_dst).reshape(num_steps * e_pad // B, B)
    ones_row = jnp.ones((B,), jnp.float32)

    deg_kernel = _make_degree_kernel(n_acc2, deg_rows_per_tile)
    degp = deg_kernel(ddst2d, ones_row)
    degp = degp.reshape(NC, num_steps, n_acc)

    edge_kernel = _make_edge_kernel(n, d, n_acc, ept // B)
    block_rows = 2000

    dps = [(degp[0, s, :n].reshape(n, 1), degp[1, s, :n].reshape(n, 1))
           for s in range(num_steps)]

    y = _tc_matmul_scale(x, W0, dps[0][0], dps[0][1], block_rows)
    parts = edge_kernel(src2d[0], dst2d[0], y)
    y = _tc_finalize_matmul(parts, y, dps[0][0], dps[0][1],
                            b0.reshape(1, d), W1, dps[1][0], dps[1][1],
                            block_rows)
    parts = edge_kernel(src2d[1], dst2d[1], y)
    return _tc_finalize(parts, y, dps[1][0], dps[1][1],
                        b1.reshape(1, d), False, block_rows)


# speedup vs baseline: 1.0946x; 1.0010x over previous
import functools

import jax
import jax.numpy as jnp
from jax import lax
from jax.experimental import pallas as pl
from jax.experimental.pallas import tpu as pltpu
from jax.experimental.pallas import tpu_sc as plsc

NC = 2
NS = 16
NW = NC * NS
LANES = 16
B = 128


def _sc_mesh():
    return plsc.VectorSubcoreMesh(
        core_axis_name="c", subcore_axis_name="s",
        num_cores=NC, num_subcores=NS)


def _make_degree_kernel(n_acc, rows_per_tile):
    per_tile = n_acc // NS

    @functools.partial(
        pl.kernel,
        out_type=jax.ShapeDtypeStruct((NC, 2, n_acc), jnp.float32),
        mesh=_sc_mesh(),
        scratch_types=[
            pltpu.VMEM((2 * rows_per_tile, B), jnp.int32),
            pltpu.VMEM((B,), jnp.float32),
            pltpu.VMEM((B,), jnp.float32),
            pltpu.VMEM_SHARED((n_acc,), jnp.float32),
            pltpu.VMEM_SHARED((n_acc,), jnp.float32),
            pltpu.SemaphoreType.DMA,
        ],
    )
    def deg_kernel(dst0_hbm, dst1_hbm, ones_hbm, out_hbm, didx_v,
                   ones_v, z_v, acc0_sh, acc1_sh, sem):
        cid = lax.axis_index("c")
        sid = lax.axis_index("s")
        wid = cid * NS + sid
        pltpu.async_copy(
            dst0_hbm.at[pl.ds(wid * rows_per_tile, rows_per_tile)],
            didx_v.at[pl.ds(0, rows_per_tile)], sem)
        pltpu.async_copy(
            dst1_hbm.at[pl.ds(wid * rows_per_tile, rows_per_tile)],
            didx_v.at[pl.ds(rows_per_tile, rows_per_tile)], sem)
        zf = jnp.zeros((LANES,), jnp.float32)
        for i in range(B // LANES):
            z_v[pl.ds(i * LANES, LANES)] = zf
        for acc_sh in (acc0_sh, acc1_sh):
            @pl.loop(0, per_tile // B)
            def _zero(t, acc_sh=acc_sh):
                pltpu.sync_copy(
                    z_v, acc_sh.at[pl.ds(sid * per_tile + t * B, B)])
        pltpu.sync_copy(ones_hbm, ones_v)
        pltpu.make_async_copy(
            dst0_hbm.at[pl.ds(0, rows_per_tile)],
            didx_v.at[pl.ds(0, rows_per_tile)], sem).wait()
        pltpu.make_async_copy(
            dst0_hbm.at[pl.ds(0, rows_per_tile)],
            didx_v.at[pl.ds(0, rows_per_tile)], sem).wait()
        plsc.subcore_barrier()

        for step, acc_sh in ((0, acc0_sh), (1, acc1_sh)):
            @pl.loop(0, rows_per_tile)
            def _scatter(j, step=step, acc_sh=acc_sh):
                pltpu.async_copy(
                    ones_v,
                    acc_sh.at[didx_v.at[step * rows_per_tile + j]],
                    sem, add=True)

                @pl.when(j >= 16)
                def _():
                    pltpu.make_async_copy(
                        ones_v, acc_sh.at[didx_v.at[0]], sem).wait()

            @pl.loop(0, 16)
            def _drain(j, acc_sh=acc_sh):
                pltpu.make_async_copy(ones_v, acc_sh.at[didx_v.at[0]],
                                      sem).wait()

        plsc.subcore_barrier()
        pltpu.sync_copy(acc0_sh.at[pl.ds(sid * per_tile, per_tile)],
                        out_hbm.at[cid, 0, pl.ds(sid * per_tile,
                                                 per_tile)])
        pltpu.sync_copy(acc1_sh.at[pl.ds(sid * per_tile, per_tile)],
                        out_hbm.at[cid, 1, pl.ds(sid * per_tile,
                                                 per_tile)])

    return deg_kernel


def _make_edge_kernel(n, d, n_acc, rows_per_tile):
    acc_rows = n_acc // NS
    assert rows_per_tile % 2 == 0
    win = rows_per_tile // 2

    @functools.partial(
        pl.kernel,
        out_type=jax.ShapeDtypeStruct((NC, n_acc, d), jnp.float32),
        mesh=_sc_mesh(),
        scratch_types=[
            pltpu.VMEM((win, B), jnp.int32),
            pltpu.VMEM((win, B), jnp.int32),
            pltpu.VMEM((2, B, d), jnp.float32),
            pltpu.VMEM_SHARED((n_acc, d), jnp.float32),
            pltpu.SemaphoreType.DMA,
            pltpu.SemaphoreType.DMA,
            pltpu.SemaphoreType.DMA,
            pltpu.SemaphoreType.DMA,
        ],
    )
    def edge_kernel(src_hbm, dst_hbm, y_hbm, out_hbm, sidx_v, didx_v,
                    rows_v, acc_sh, gsem0, gsem1, ssem0, ssem1):
        cid = lax.axis_index("c")
        sid = lax.axis_index("s")
        wid = cid * NS + sid
        zf = jnp.zeros((LANES,), jnp.float32)
        for r in range(LANES):
            for i in range(d // LANES):
                rows_v[0, r, pl.ds(i * LANES, LANES)] = zf

        @pl.loop(0, acc_rows // LANES)
        def _zero(t):
            pltpu.sync_copy(
                rows_v.at[0, pl.ds(0, LANES)],
                acc_sh.at[pl.ds(sid * acc_rows + t * LANES, LANES)])

        plsc.subcore_barrier()

        gsems = (gsem0, gsem1)
        ssems = (ssem0, ssem1)
        for half in range(2):
            base = wid * rows_per_tile + half * win
            pltpu.async_copy(src_hbm.at[pl.ds(base, win)], sidx_v,
                             gsem0).wait()
            pltpu.async_copy(dst_hbm.at[pl.ds(base, win)], didx_v,
                             gsem0).wait()
            pltpu.async_copy(y_hbm.at[sidx_v.at[0]], rows_v.at[0],
                             gsem0)

            @pl.loop(0, win)
            def _edges(j):
                b = lax.rem(j, 2)
                for k in range(2):
                    @pl.when(b == k)
                    def _(k=k):
                        ko = 1 - k

                        @pl.when(j >= 1)
                        def _():
                            pltpu.make_async_copy(
                                rows_v.at[ko],
                                acc_sh.at[didx_v.at[j - 1]],
                                ssems[ko]).wait()

                        @pl.when(j + 1 < win)
                        def _():
                            pltpu.async_copy(y_hbm.at[sidx_v.at[j + 1]],
                                             rows_v.at[ko], gsems[ko])
                        pltpu.make_async_copy(
                            y_hbm.at[sidx_v.at[j]], rows_v.at[k],
                            gsems[k]).wait()
                        pltpu.async_copy(rows_v.at[k],
                                         acc_sh.at[didx_v.at[j]],
                                         ssems[k], add=True)

            lk = (win - 1) % 2
            pltpu.make_async_copy(rows_v.at[lk],
                                  acc_sh.at[didx_v.at[win - 1]],
                                  ssems[lk]).wait()
        plsc.subcore_barrier()
        pltpu.sync_copy(acc_sh.at[pl.ds(sid * acc_rows, acc_rows)],
                        out_hbm.at[cid, pl.ds(sid * acc_rows, acc_rows)])

    return edge_kernel


def _tc_matmul_scale(h, w, dp0, dp1, block_rows):
    n, d = h.shape

    def body(h_ref, w_ref, dp0_ref, dp1_ref, y_ref):
        dis = lax.rsqrt(1.0 + dp0_ref[...] + dp1_ref[...])
        xw = jnp.dot(h_ref[...], w_ref[...],
                     preferred_element_type=jnp.float32)
        y_ref[...] = xw * dis

    return pl.pallas_call(
        body,
        grid=(n // block_rows,),
        in_specs=[
            pl.BlockSpec((block_rows, d), lambda i: (i, 0)),
            pl.BlockSpec((d, d), lambda i: (0, 0)),
            pl.BlockSpec((block_rows, 1), lambda i: (i, 0)),
            pl.BlockSpec((block_rows, 1), lambda i: (i, 0)),
        ],
        out_specs=pl.BlockSpec((block_rows, d), lambda i: (i, 0)),
        out_shape=jax.ShapeDtypeStruct((n, d), jnp.float32),
    )(h, w, dp0, dp1)


def _tc_finalize_matmul(parts, y, dp0, dp1, b2d, w_next, dpn0, dpn1,
                        block_rows):
    n, d = y.shape

    def body(p0_ref, p1_ref, y_ref, dp0_ref, dp1_ref, b_ref, w_ref,
             dpn0_ref, dpn1_ref, o_ref):
        dis = lax.rsqrt(1.0 + dp0_ref[...] + dp1_ref[...])
        t = (p0_ref[0] + p1_ref[0] + y_ref[...]) * dis + b_ref[...]
        t = jnp.maximum(t, 0.0)
        nrm = jnp.sqrt(jnp.sum(t * t, axis=1, keepdims=True))
        h = t / jnp.maximum(nrm, 1e-12)
        disn = lax.rsqrt(1.0 + dpn0_ref[...] + dpn1_ref[...])
        o_ref[...] = jnp.dot(h, w_ref[...],
                             preferred_element_type=jnp.float32) * disn

    row_spec = pl.BlockSpec((block_rows, d), lambda i: (i, 0))
    col_spec = pl.BlockSpec((block_rows, 1), lambda i: (i, 0))
    part0_spec = pl.BlockSpec((1, block_rows, d), lambda i: (0, i, 0))
    part1_spec = pl.BlockSpec((1, block_rows, d), lambda i: (1, i, 0))
    return pl.pallas_call(
        body,
        grid=(n // block_rows,),
        in_specs=[
            part0_spec, part1_spec, row_spec, col_spec, col_spec,
            pl.BlockSpec((1, d), lambda i: (0, 0)),
            pl.BlockSpec((d, d), lambda i: (0, 0)),
            col_spec, col_spec,
        ],
        out_specs=row_spec,
        out_shape=jax.ShapeDtypeStruct((n, d), jnp.float32),
    )(parts, parts, y, dp0, dp1, b2d, w_next, dpn0, dpn1)


def _tc_finalize(parts, y, dp0, dp1, b2d, relu, block_rows):
    n, d = y.shape

    def body(p0_ref, p1_ref, y_ref, dp0_ref, dp1_ref, b_ref, o_ref):
        dis = lax.rsqrt(1.0 + dp0_ref[...] + dp1_ref[...])
        t = (p0_ref[0] + p1_ref[0] + y_ref[...]) * dis + b_ref[...]
        if relu:
            t = jnp.maximum(t, 0.0)
        nrm = jnp.sqrt(jnp.sum(t * t, axis=1, keepdims=True))
        o_ref[...] = t / jnp.maximum(nrm, 1e-12)

    return pl.pallas_call(
        body,
        grid=(n // block_rows,),
        in_specs=[
            pl.BlockSpec((1, block_rows, d), lambda i: (0, i, 0)),
            pl.BlockSpec((1, block_rows, d), lambda i: (1, i, 0)),
            pl.BlockSpec((block_rows, d), lambda i: (i, 0)),
            pl.BlockSpec((block_rows, 1), lambda i: (i, 0)),
            pl.BlockSpec((block_rows, 1), lambda i: (i, 0)),
            pl.BlockSpec((1, d), lambda i: (0, 0)),
        ],
        out_specs=pl.BlockSpec((block_rows, d), lambda i: (i, 0)),
        out_shape=jax.ShapeDtypeStruct((n, d), jnp.float32),
    )(parts, parts, y, dp0, dp1, b2d)


def kernel(x, edge_index_list, W0, b0, W1, b1):
    n, d = x.shape
    num_steps = edge_index_list.shape[0]
    e = edge_index_list.shape[2]

    n_acc = ((n + B) + 1023) // 1024 * 1024
    ept = -(-e // (NW * 1024)) * 1024
    e_pad = NW * ept
    pad_len = e_pad - e

    idx_dtype = edge_index_list.dtype
    pad_cycle = jnp.arange(pad_len, dtype=idx_dtype) % B
    pad_src = pad_cycle
    pad_dst = n + pad_cycle

    src2d = []
    dst2d = []
    for s in range(num_steps):
        src_s = jnp.concatenate([edge_index_list[s, 0], pad_src])
        dst_s = jnp.concatenate([edge_index_list[s, 1], pad_dst])
        src2d.append(src_s.reshape(e_pad // B, B))
        dst2d.append(dst_s.reshape(e_pad // B, B))

    ones_row = jnp.ones((B,), jnp.float32)
    deg_kernel = _make_degree_kernel(n_acc, ept // B)
    degp = deg_kernel(dst2d[0], dst2d[1], ones_row)

    edge_kernel = _make_edge_kernel(n, d, n_acc, ept // B)
    block_rows = 2000

    dps = [(degp[0, s, :n].reshape(n, 1), degp[1, s, :n].reshape(n, 1))
           for s in range(num_steps)]

    y = _tc_matmul_scale(x, W0, dps[0][0], dps[0][1], block_rows)
    parts = edge_kernel(src2d[0], dst2d[0], y)
    y = _tc_finalize_matmul(parts, y, dps[0][0], dps[0][1],
                            b0.reshape(1, d), W1, dps[1][0], dps[1][1],
                            block_rows)
    parts = edge_kernel(src2d[1], dst2d[1], y)
    return _tc_finalize(parts, y, dps[1][0], dps[1][1],
                        b1.reshape(1, d), False, block_rows)
